# Initial kernel scaffold; baseline (speedup 1.0000x reference)
import jax
import jax.numpy as jnp
from jax.experimental import pallas as pl


def kernel(z, embedding, W_down, W_up):
    raise NotImplementedError("write your pallas kernel here")


# speedup vs baseline: 1.0926x; 1.0926x over previous
import functools

import jax
import jax.numpy as jnp
from jax import lax
from jax.experimental import pallas as pl
from jax.experimental.pallas import tpu as pltpu
from jax.experimental.pallas import tpu_sc as plsc

B, T, DIN = 16, 1024, 512
D = 256
K = 8192
N = B * T

RBLK = 1024
KBLK = 1024


def _dist_argmin_body(z_ref, wd_ref, et_ref, ze_ref, code_ref, mind_ref):
    k = pl.program_id(1)

    @pl.when(k == 0)
    def _():
        ze_ref[...] = lax.dot_general(
            z_ref[...].astype(jnp.bfloat16), wd_ref[...].astype(jnp.bfloat16),
            (((1,), (1,)), ((), ())),
            preferred_element_type=jnp.float32)

    ze = ze_ref[...]
    et = et_ref[...]
    dot = lax.dot_general(ze.astype(jnp.bfloat16), et.astype(jnp.bfloat16),
                          (((1,), (0,)), ((), ())),
                          preferred_element_type=jnp.float32)
    en = jnp.sum(et * et, axis=0, keepdims=True)
    zn = jnp.sum(ze * ze, axis=1, keepdims=True)
    d = (zn - 2.0 * dot) + en

    tmin = jnp.min(d, axis=1, keepdims=True)
    iot = lax.broadcasted_iota(jnp.int32, d.shape, 1)
    targ = jnp.min(jnp.where(d == tmin, iot, K), axis=1, keepdims=True) \
        + k * KBLK

    @pl.when(k == 0)
    def _():
        mind_ref[...] = tmin
        code_ref[...] = targ

    @pl.when(k > 0)
    def _():
        better = tmin < mind_ref[...]
        code_ref[...] = jnp.where(better, targ, code_ref[...])
        mind_ref[...] = jnp.where(better, tmin, mind_ref[...])


def _upproj_body(e_ref, wu_ref, out_ref):
    out_ref[...] = lax.dot_general(
        e_ref[...].astype(jnp.bfloat16), wu_ref[...].astype(jnp.bfloat16),
        (((1,), (1,)), ((), ())),
        preferred_element_type=jnp.float32)


_NC, _NS = 2, 16
_NW = _NC * _NS
_BPW = N // _NW
_CHUNK = 128


def _make_gather():
    mesh = plsc.VectorSubcoreMesh(core_axis_name="c", subcore_axis_name="s")

    @functools.partial(
        pl.kernel,
        mesh=mesh,
        out_type=jax.ShapeDtypeStruct((N, DIN), jnp.float32),
        scratch_types=[
            pltpu.VMEM((_CHUNK,), jnp.int32),
            pltpu.VMEM((_CHUNK, DIN), jnp.float32),
            pltpu.SemaphoreType.DMA,
        ],
    )
    def gather(eup_hbm, code_hbm, out_hbm, idx_v, rows_v, sem):
        wid = lax.axis_index("s") * _NC + lax.axis_index("c")
        base = wid * _BPW
        for c in range(_BPW // _CHUNK):
            off = base + c * _CHUNK
            pltpu.sync_copy(code_hbm.at[pl.ds(off, _CHUNK)], idx_v)
            pltpu.async_copy(eup_hbm.at[idx_v], rows_v, sem).wait()
            pltpu.sync_copy(rows_v, out_hbm.at[pl.ds(off, _CHUNK)])

    return gather


def kernel(z, embedding, W_down, W_up):
    z_flat = z.reshape(N, DIN)
    e_t = embedding.T

    ze_flat, code2d, mind = pl.pallas_call(
        _dist_argmin_body,
        grid=(N // RBLK, K // KBLK),
        in_specs=[
            pl.BlockSpec((RBLK, DIN), lambda r, k: (r, 0)),
            pl.BlockSpec((D, DIN), lambda r, k: (0, 0)),
            pl.BlockSpec((D, KBLK), lambda r, k: (0, k)),
        ],
        out_specs=[
            pl.BlockSpec((RBLK, D), lambda r, k: (r, 0)),
            pl.BlockSpec((RBLK, 1), lambda r, k: (r, 0)),
            pl.BlockSpec((RBLK, 1), lambda r, k: (r, 0)),
        ],
        out_shape=[
            jax.ShapeDtypeStruct((N, D), jnp.float32),
            jax.ShapeDtypeStruct((N, 1), jnp.int32),
            jax.ShapeDtypeStruct((N, 1), jnp.float32),
        ],
    )(z_flat, W_down, e_t)

    eup = pl.pallas_call(
        _upproj_body,
        grid=(K // RBLK,),
        in_specs=[
            pl.BlockSpec((RBLK, D), lambda r: (r, 0)),
            pl.BlockSpec((DIN, D), lambda r: (0, 0)),
        ],
        out_specs=pl.BlockSpec((RBLK, DIN), lambda r: (r, 0)),
        out_shape=jax.ShapeDtypeStruct((K, DIN), jnp.float32),
    )(embedding, W_up)

    code_flat = code2d[:, 0]
    zq_out_flat = _make_gather()(eup, code_flat)

    loss = mind[:, 0].reshape(B, T).mean(axis=1) / D
    return (
        zq_out_flat.reshape(B, T, DIN),
        loss,
        loss,
        code_flat.reshape(B, T),
        ze_flat.reshape(B, T, D),
    )


# speedup vs baseline: 1.2630x; 1.1560x over previous
import functools

import jax
import jax.numpy as jnp
from jax import lax
from jax.experimental import pallas as pl
from jax.experimental.pallas import tpu as pltpu
from jax.experimental.pallas import tpu_sc as plsc

B, T, DIN = 16, 1024, 512
D = 256
K = 8192
N = B * T

RBLK = 1024
KBLK = 2048


def _dist_argmin_body(z_ref, wd_ref, et_ref, ze_ref, code_ref, mind_ref):
    k = pl.program_id(1)

    @pl.when(k == 0)
    def _():
        ze_ref[...] = lax.dot_general(
            z_ref[...].astype(jnp.bfloat16), wd_ref[...].astype(jnp.bfloat16),
            (((1,), (1,)), ((), ())),
            preferred_element_type=jnp.float32)

    ze = ze_ref[...]
    et = et_ref[...]
    dot = lax.dot_general(ze.astype(jnp.bfloat16), et.astype(jnp.bfloat16),
                          (((1,), (0,)), ((), ())),
                          preferred_element_type=jnp.float32)
    en = jnp.sum(et * et, axis=0, keepdims=True)
    d = en - 2.0 * dot

    tmin = jnp.min(d, axis=1, keepdims=True)
    iot = lax.broadcasted_iota(jnp.int32, d.shape, 1)
    targ = jnp.min(jnp.where(d == tmin, iot, K), axis=1, keepdims=True) \
        + k * KBLK

    @pl.when(k == 0)
    def _():
        mind_ref[...] = tmin
        code_ref[...] = targ

    @pl.when(k > 0)
    def _():
        better = tmin < mind_ref[...]
        code_ref[...] = jnp.where(better, targ, code_ref[...])
        mind_ref[...] = jnp.where(better, tmin, mind_ref[...])

    @pl.when(k == (K // KBLK) - 1)
    def _():
        zn = jnp.sum(ze * ze, axis=1, keepdims=True)
        mind_ref[...] = mind_ref[...] + zn


def _upproj_body(e_ref, wu_ref, out_ref):
    out_ref[...] = lax.dot_general(
        e_ref[...].astype(jnp.bfloat16), wu_ref[...].astype(jnp.bfloat16),
        (((1,), (1,)), ((), ())),
        preferred_element_type=jnp.float32)


_NC, _NS = 2, 16
_NW = _NC * _NS
_BPW = N // _NW
_CHUNK = 128


def _make_gather():
    mesh = plsc.VectorSubcoreMesh(core_axis_name="c", subcore_axis_name="s")

    @functools.partial(
        pl.kernel,
        mesh=mesh,
        out_type=jax.ShapeDtypeStruct((N, DIN), jnp.float32),
        scratch_types=[
            pltpu.VMEM((_CHUNK,), jnp.int32),
            pltpu.VMEM((_CHUNK, DIN), jnp.float32),
            pltpu.SemaphoreType.DMA,
        ],
    )
    def gather(eup_hbm, code_hbm, out_hbm, idx_v, rows_v, sem):
        wid = lax.axis_index("s") * _NC + lax.axis_index("c")
        base = wid * _BPW
        for c in range(_BPW // _CHUNK):
            off = base + c * _CHUNK
            pltpu.sync_copy(code_hbm.at[pl.ds(off, _CHUNK)], idx_v)
            pltpu.async_copy(eup_hbm.at[idx_v], rows_v, sem).wait()
            pltpu.sync_copy(rows_v, out_hbm.at[pl.ds(off, _CHUNK)])

    return gather


def kernel(z, embedding, W_down, W_up):
    z_flat = z.reshape(N, DIN)
    e_t = embedding.T

    ze_flat, code2d, mind = pl.pallas_call(
        _dist_argmin_body,
        grid=(N // RBLK, K // KBLK),
        in_specs=[
            pl.BlockSpec((RBLK, DIN), lambda r, k: (r, 0)),
            pl.BlockSpec((D, DIN), lambda r, k: (0, 0)),
            pl.BlockSpec((D, KBLK), lambda r, k: (0, k)),
        ],
        out_specs=[
            pl.BlockSpec((RBLK, D), lambda r, k: (r, 0)),
            pl.BlockSpec((RBLK, 1), lambda r, k: (r, 0)),
            pl.BlockSpec((RBLK, 1), lambda r, k: (r, 0)),
        ],
        out_shape=[
            jax.ShapeDtypeStruct((N, D), jnp.float32),
            jax.ShapeDtypeStruct((N, 1), jnp.int32),
            jax.ShapeDtypeStruct((N, 1), jnp.float32),
        ],
    )(z_flat, W_down, e_t)

    eup = pl.pallas_call(
        _upproj_body,
        grid=(K // RBLK,),
        in_specs=[
            pl.BlockSpec((RBLK, D), lambda r: (r, 0)),
            pl.BlockSpec((DIN, D), lambda r: (0, 0)),
        ],
        out_specs=pl.BlockSpec((RBLK, DIN), lambda r: (r, 0)),
        out_shape=jax.ShapeDtypeStruct((K, DIN), jnp.float32),
    )(embedding, W_up)

    code_flat = code2d[:, 0]
    zq_out_flat = _make_gather()(eup, code_flat)

    loss = mind[:, 0].reshape(B, T).mean(axis=1) / D
    return (
        zq_out_flat.reshape(B, T, DIN),
        loss,
        loss,
        code_flat.reshape(B, T),
        ze_flat.reshape(B, T, D),
    )


# speedup vs baseline: 1.3384x; 1.0597x over previous
import functools

import jax
import jax.numpy as jnp
from jax import lax
from jax.experimental import pallas as pl
from jax.experimental.pallas import tpu as pltpu
from jax.experimental.pallas import tpu_sc as plsc

B, T, DIN = 16, 1024, 512
D = 256
K = 8192
N = B * T

RBLK = 1024
KBLK = 4096


def _dist_argmin_body(z_ref, wd_ref, et_ref, ze_ref, code_ref, mind_ref):
    k = pl.program_id(1)

    @pl.when(k == 0)
    def _():
        ze_ref[...] = lax.dot_general(
            z_ref[...].astype(jnp.bfloat16), wd_ref[...].astype(jnp.bfloat16),
            (((1,), (1,)), ((), ())),
            preferred_element_type=jnp.float32)

    ze = ze_ref[...]
    et = et_ref[...]
    dot = lax.dot_general(ze.astype(jnp.bfloat16), et.astype(jnp.bfloat16),
                          (((1,), (0,)), ((), ())),
                          preferred_element_type=jnp.float32)
    en = jnp.sum(et * et, axis=0, keepdims=True)
    d = en - 2.0 * dot

    tmin = jnp.min(d, axis=1, keepdims=True)
    iot = lax.broadcasted_iota(jnp.int32, d.shape, 1)
    targ = jnp.min(jnp.where(d == tmin, iot, K), axis=1, keepdims=True) \
        + k * KBLK

    @pl.when(k == 0)
    def _():
        mind_ref[...] = tmin
        code_ref[...] = targ

    @pl.when(k > 0)
    def _():
        better = tmin < mind_ref[...]
        code_ref[...] = jnp.where(better, targ, code_ref[...])
        mind_ref[...] = jnp.where(better, tmin, mind_ref[...])

    @pl.when(k == (K // KBLK) - 1)
    def _():
        zn = jnp.sum(ze * ze, axis=1, keepdims=True)
        mind_ref[...] = mind_ref[...] + zn


def _upproj_body(e_ref, wu_ref, out_ref):
    out_ref[...] = lax.dot_general(
        e_ref[...].astype(jnp.bfloat16), wu_ref[...].astype(jnp.bfloat16),
        (((1,), (1,)), ((), ())),
        preferred_element_type=jnp.float32)


_NC, _NS = 2, 16
_NW = _NC * _NS
_BPW = N // _NW
_CHUNK = 128


def _make_gather():
    mesh = plsc.VectorSubcoreMesh(core_axis_name="c", subcore_axis_name="s")

    @functools.partial(
        pl.kernel,
        mesh=mesh,
        out_type=jax.ShapeDtypeStruct((N, DIN), jnp.float32),
        scratch_types=[
            pltpu.VMEM((_CHUNK,), jnp.int32),
            pltpu.VMEM((_CHUNK, DIN), jnp.float32),
            pltpu.SemaphoreType.DMA,
        ],
    )
    def gather(eup_hbm, code_hbm, out_hbm, idx_v, rows_v, sem):
        wid = lax.axis_index("s") * _NC + lax.axis_index("c")
        base = wid * _BPW
        for c in range(_BPW // _CHUNK):
            off = base + c * _CHUNK
            pltpu.sync_copy(code_hbm.at[pl.ds(off, _CHUNK)], idx_v)
            pltpu.async_copy(eup_hbm.at[idx_v], rows_v, sem).wait()
            pltpu.sync_copy(rows_v, out_hbm.at[pl.ds(off, _CHUNK)])

    return gather


def kernel(z, embedding, W_down, W_up):
    z_flat = z.reshape(N, DIN)
    e_t = embedding.T

    ze_flat, code2d, mind = pl.pallas_call(
        _dist_argmin_body,
        grid=(N // RBLK, K // KBLK),
        in_specs=[
            pl.BlockSpec((RBLK, DIN), lambda r, k: (r, 0)),
            pl.BlockSpec((D, DIN), lambda r, k: (0, 0)),
            pl.BlockSpec((D, KBLK), lambda r, k: (0, k)),
        ],
        out_specs=[
            pl.BlockSpec((RBLK, D), lambda r, k: (r, 0)),
            pl.BlockSpec((RBLK, 1), lambda r, k: (r, 0)),
            pl.BlockSpec((RBLK, 1), lambda r, k: (r, 0)),
        ],
        out_shape=[
            jax.ShapeDtypeStruct((N, D), jnp.float32),
            jax.ShapeDtypeStruct((N, 1), jnp.int32),
            jax.ShapeDtypeStruct((N, 1), jnp.float32),
        ],
    )(z_flat, W_down, e_t)

    eup = pl.pallas_call(
        _upproj_body,
        grid=(K // RBLK,),
        in_specs=[
            pl.BlockSpec((RBLK, D), lambda r: (r, 0)),
            pl.BlockSpec((DIN, D), lambda r: (0, 0)),
        ],
        out_specs=pl.BlockSpec((RBLK, DIN), lambda r: (r, 0)),
        out_shape=jax.ShapeDtypeStruct((K, DIN), jnp.float32),
    )(embedding, W_up)

    code_flat = code2d[:, 0]
    zq_out_flat = _make_gather()(eup, code_flat)

    loss = mind[:, 0].reshape(B, T).mean(axis=1) / D
    return (
        zq_out_flat.reshape(B, T, DIN),
        loss,
        loss,
        code_flat.reshape(B, T),
        ze_flat.reshape(B, T, D),
    )


# speedup vs baseline: 1.3723x; 1.0253x over previous
import functools

import jax
import jax.numpy as jnp
from jax import lax
from jax.experimental import pallas as pl
from jax.experimental.pallas import tpu as pltpu
from jax.experimental.pallas import tpu_sc as plsc

B, T, DIN = 16, 1024, 512
D = 256
K = 8192
N = B * T

RBLK = 2048
KBLK = 4096


def _dist_argmin_body(z_ref, wd_ref, et_ref, ze_ref, code_ref, mind_ref):
    k = pl.program_id(1)

    @pl.when(k == 0)
    def _():
        ze_ref[...] = lax.dot_general(
            z_ref[...].astype(jnp.bfloat16), wd_ref[...].astype(jnp.bfloat16),
            (((1,), (1,)), ((), ())),
            preferred_element_type=jnp.float32)

    ze = ze_ref[...]
    et = et_ref[...]
    dot = lax.dot_general(ze.astype(jnp.bfloat16), et.astype(jnp.bfloat16),
                          (((1,), (0,)), ((), ())),
                          preferred_element_type=jnp.float32)
    en = jnp.sum(et * et, axis=0, keepdims=True)
    d = en - 2.0 * dot

    tmin = jnp.min(d, axis=1, keepdims=True)
    iot = lax.broadcasted_iota(jnp.int32, d.shape, 1)
    targ = jnp.min(jnp.where(d == tmin, iot, K), axis=1, keepdims=True) \
        + k * KBLK

    @pl.when(k == 0)
    def _():
        mind_ref[...] = tmin
        code_ref[...] = targ

    @pl.when(k > 0)
    def _():
        better = tmin < mind_ref[...]
        code_ref[...] = jnp.where(better, targ, code_ref[...])
        mind_ref[...] = jnp.where(better, tmin, mind_ref[...])

    @pl.when(k == (K // KBLK) - 1)
    def _():
        zn = jnp.sum(ze * ze, axis=1, keepdims=True)
        mind_ref[...] = mind_ref[...] + zn


def _upproj_body(e_ref, wu_ref, out_ref):
    out_ref[...] = lax.dot_general(
        e_ref[...].astype(jnp.bfloat16), wu_ref[...].astype(jnp.bfloat16),
        (((1,), (1,)), ((), ())),
        preferred_element_type=jnp.float32)


_NC, _NS = 2, 16
_NW = _NC * _NS
_BPW = N // _NW
_CHUNK = 128


def _make_gather():
    mesh = plsc.VectorSubcoreMesh(core_axis_name="c", subcore_axis_name="s")

    @functools.partial(
        pl.kernel,
        mesh=mesh,
        out_type=jax.ShapeDtypeStruct((N, DIN), jnp.float32),
        scratch_types=[
            pltpu.VMEM((_CHUNK,), jnp.int32),
            pltpu.VMEM((_CHUNK, DIN), jnp.float32),
            pltpu.SemaphoreType.DMA,
        ],
    )
    def gather(eup_hbm, code_hbm, out_hbm, idx_v, rows_v, sem):
        wid = lax.axis_index("s") * _NC + lax.axis_index("c")
        base = wid * _BPW
        for c in range(_BPW // _CHUNK):
            off = base + c * _CHUNK
            pltpu.sync_copy(code_hbm.at[pl.ds(off, _CHUNK)], idx_v)
            pltpu.async_copy(eup_hbm.at[idx_v], rows_v, sem).wait()
            pltpu.sync_copy(rows_v, out_hbm.at[pl.ds(off, _CHUNK)])

    return gather


def kernel(z, embedding, W_down, W_up):
    z_flat = z.reshape(N, DIN)
    e_t = embedding.T

    ze_flat, code2d, mind = pl.pallas_call(
        _dist_argmin_body,
        grid=(N // RBLK, K // KBLK),
        in_specs=[
            pl.BlockSpec((RBLK, DIN), lambda r, k: (r, 0)),
            pl.BlockSpec((D, DIN), lambda r, k: (0, 0)),
            pl.BlockSpec((D, KBLK), lambda r, k: (0, k)),
        ],
        out_specs=[
            pl.BlockSpec((RBLK, D), lambda r, k: (r, 0)),
            pl.BlockSpec((RBLK, 1), lambda r, k: (r, 0)),
            pl.BlockSpec((RBLK, 1), lambda r, k: (r, 0)),
        ],
        out_shape=[
            jax.ShapeDtypeStruct((N, D), jnp.float32),
            jax.ShapeDtypeStruct((N, 1), jnp.int32),
            jax.ShapeDtypeStruct((N, 1), jnp.float32),
        ],
    )(z_flat, W_down, e_t)

    eup = pl.pallas_call(
        _upproj_body,
        grid=(K // RBLK,),
        in_specs=[
            pl.BlockSpec((RBLK, D), lambda r: (r, 0)),
            pl.BlockSpec((DIN, D), lambda r: (0, 0)),
        ],
        out_specs=pl.BlockSpec((RBLK, DIN), lambda r: (r, 0)),
        out_shape=jax.ShapeDtypeStruct((K, DIN), jnp.float32),
    )(embedding, W_up)

    code_flat = code2d[:, 0]
    zq_out_flat = _make_gather()(eup, code_flat)

    loss = mind[:, 0].reshape(B, T).mean(axis=1) / D
    return (
        zq_out_flat.reshape(B, T, DIN),
        loss,
        loss,
        code_flat.reshape(B, T),
        ze_flat.reshape(B, T, D),
    )
